# baseline (speedup 1.0000x reference)
import functools

import jax
import jax.numpy as jnp
from jax import lax
from jax.experimental import pallas as pl
from jax.experimental.pallas import tpu as pltpu
from jax.experimental.pallas import tpu_sc as plsc

EMBED = 768
HEADS = 12
HD = EMBED // HEADS
NE = 96
S = 4096
B = 2
QBLK = 256
NBLK = S // QBLK

ROWS = B * S
NSLICE = 4
SR = ROWS // NSLICE
NBLK_SL = SR // QBLK
NC = 2
NS = 16
NTILES = NC * NS
RPT = SR // NTILES
LN = 16
NCHUNK = S // LN


def _make_tc_body(row0):
    def _tc_body(keys_ref, q_ref, wk_ref, wq_ref, e_ref, t_ref, nt_ref,
                 cge_ref, nodes_ref, kph_ref, kpl_ref):
        j = pl.program_id(0)

        @pl.when(j == 0)
        def _project_keys():
            for h in range(HEADS):
                kh = keys_ref[0, :, h * HD:(h + 1) * HD]
                kp = lax.dot_general(
                    kh, wk_ref[...], (((1,), (1,)), ((), ())),
                    preferred_element_type=jnp.float32)
                kp_hi = kp.astype(jnp.bfloat16)
                kph_ref[:, h * HD:(h + 1) * HD] = kp_hi
                kpl_ref[:, h * HD:(h + 1) * HD] = (
                    kp - kp_hi.astype(jnp.float32)).astype(jnp.bfloat16)

        qblk = q_ref[0]
        qp = jnp.concatenate(
            [lax.dot_general(qblk[:, h * HD:(h + 1) * HD], wq_ref[...],
                             (((1,), (1,)), ((), ())),
                             preferred_element_type=jnp.float32)
             for h in range(HEADS)], axis=1)
        qp_hi = qp.astype(jnp.bfloat16)
        qp_lo = (qp - qp_hi.astype(jnp.float32)).astype(jnp.bfloat16)
        dims = (((1,), (1,)), ((), ()))
        e = lax.dot_general(qp_hi, kph_ref[...], dims,
                            preferred_element_type=jnp.float32)
        e = e + lax.dot_general(qp_hi, kpl_ref[...], dims,
                                preferred_element_type=jnp.float32)
        e = e + lax.dot_general(qp_lo, kph_ref[...], dims,
                                preferred_element_type=jnp.float32)
        e_ref[...] = e

        bits = lax.bitcast_convert_type(e, jnp.int32)
        skey = jnp.where(bits < 0, bits ^ jnp.int32(0x7FFFFFFF), bits)

        lo = jnp.full((QBLK, 1), jnp.iinfo(jnp.int32).min, dtype=jnp.int32)
        hi = jnp.full((QBLK, 1), jnp.iinfo(jnp.int32).max, dtype=jnp.int32)

        def bs_body(_, carry):
            lo, hi = carry
            extra = ((lo & 1) + (hi & 1) + 1) >> 1
            mid = (lo >> 1) + (hi >> 1) + extra
            cnt = jnp.sum((skey >= mid).astype(jnp.int32), axis=1,
                          keepdims=True)
            ge = cnt >= NE
            return jnp.where(ge, mid, lo), jnp.where(ge, hi, mid - 1)

        lo, hi = lax.fori_loop(0, 32, bs_body, (lo, hi))
        tbits = jnp.where(lo < 0, lo ^ jnp.int32(0x7FFFFFFF), lo)
        t = lax.bitcast_convert_type(tbits, jnp.float32)
        t_ref[...] = t

        strict = jnp.sum((e > t).astype(jnp.int32), axis=1, keepdims=True)
        nt_ref[...] = NE - strict
        cge_ref[...] = jnp.sum((e >= t).astype(jnp.int32), axis=1,
                               keepdims=True)

        nodes_ref[...] = (lax.broadcasted_iota(jnp.int32, (QBLK, NE), 0)
                          + (row0 + j * QBLK)).astype(jnp.float32)

    return _tc_body


def _tc_energy_threshold(keys, query, wk, wq, bs, hs):
    boff = hs * NBLK_SL
    return pl.pallas_call(
        _make_tc_body(hs * SR),
        grid=(NBLK_SL,),
        in_specs=[
            pl.BlockSpec((1, S, EMBED), lambda j: (bs, 0, 0)),
            pl.BlockSpec((1, QBLK, EMBED), lambda j: (bs, boff + j, 0)),
            pl.BlockSpec((HD, HD), lambda j: (0, 0)),
            pl.BlockSpec((HD, HD), lambda j: (0, 0)),
        ],
        out_specs=[
            pl.BlockSpec((QBLK, S), lambda j: (j, 0)),
            pl.BlockSpec((QBLK, 1), lambda j: (j, 0)),
            pl.BlockSpec((QBLK, 1), lambda j: (j, 0)),
            pl.BlockSpec((QBLK, 1), lambda j: (j, 0)),
            pl.BlockSpec((QBLK, NE), lambda j: (j, 0)),
        ],
        out_shape=[
            jax.ShapeDtypeStruct((SR, S), jnp.float32),
            jax.ShapeDtypeStruct((SR, 1), jnp.float32),
            jax.ShapeDtypeStruct((SR, 1), jnp.int32),
            jax.ShapeDtypeStruct((SR, 1), jnp.int32),
            jax.ShapeDtypeStruct((SR, NE), jnp.float32),
        ],
        scratch_shapes=[pltpu.VMEM((S, EMBED), jnp.bfloat16),
                        pltpu.VMEM((S, EMBED), jnp.bfloat16)],
    )(keys, query, wk, wq)


NCB = S // 128
NG = RPT // 8


def _sc_body(e_hbm, t_hbm, nt_hbm, cge_hbm, out_hbm, rowbuf, tbuf, ntbuf,
             cgebuf, sbuf, sem0, sem1, semo0, semo1):
    c = lax.axis_index("c")
    s = lax.axis_index("s")
    wid = s * NC + c
    base = wid * RPT

    pltpu.sync_copy(t_hbm.at[pl.ds(base, RPT)], tbuf.at[pl.ds(0, RPT)])
    pltpu.sync_copy(nt_hbm.at[pl.ds(base, RPT)], ntbuf.at[pl.ds(0, RPT)])
    pltpu.sync_copy(cge_hbm.at[pl.ds(base, RPT)], cgebuf.at[pl.ds(0, RPT)])

    idx0f = lax.iota(jnp.int32, 16).astype(jnp.float32)

    def dma_group(g, p, sem):
        pltpu.async_copy(e_hbm.at[pl.ds(base + g * 8, 8)], rowbuf.at[p], sem)

    def wait_group(g, p, sem):
        pltpu.make_async_copy(e_hbm.at[pl.ds(base + g * 8, 8)],
                              rowbuf.at[p], sem).wait()

    dma_group(0, 0, sem0)
    dma_group(1, 1, sem1)

    def process_group(g, p, sem, semo):
        rows0 = base + g * 8
        wait_group(g, p, sem)

        @pl.when(g >= 2)
        def _drain_out():
            for r8 in range(8):
                pltpu.make_async_copy(
                    sbuf.at[p, r8, pl.ds(0, NE)],
                    out_hbm.at[pl.ds((rows0 - 16 + r8) * NE, NE)],
                    semo).wait()

        r0 = g * 8
        tvs = [jnp.full((16,), tbuf[pl.ds(r0 + r8, 16)][0],
                        dtype=jnp.float32) for r8 in range(8)]
        cges = [cgebuf[pl.ds(r0 + r8, 16)][0] for r8 in range(8)]
        gfast = cges[0] == NE
        for r8 in range(1, 8):
            gfast = gfast & (cges[r8] == NE)

        def _chunk_addr(i, r8):
            return r8, i * LN, idx0f + (i * LN).astype(jnp.float32)

        @pl.when(gfast)
        def _fast():
            def chunk_body(i, curs):
                out = []
                for r8 in range(8):
                    ri, joff, fidx = _chunk_addr(i, r8)
                    cur = curs[r8]
                    v = rowbuf[p, ri, pl.ds(joff, LN)]
                    m = v >= tvs[r8]
                    plsc.store_compressed(sbuf.at[p, r8, pl.ds(cur, LN)],
                                          fidx, mask=m)
                    out.append(cur + plsc.all_reduce_population_count(m)[0])
                return tuple(out)

            lax.fori_loop(0, NCHUNK, chunk_body,
                          tuple(jnp.zeros((), jnp.int32) for _ in range(8)))

        @pl.when(jnp.logical_not(gfast))
        def _slow():
            for r8 in range(8):
                ntv = jnp.full((16,), ntbuf[pl.ds(r0 + r8, 16)][0],
                               dtype=jnp.int32)

                def chunk_body(i, carry):
                    selc, tiec = carry
                    ri, joff, fidx = _chunk_addr(i, r8)
                    v = rowbuf[p, ri, pl.ds(joff, LN)]
                    m_gt = v > tvs[r8]
                    m_eq = v == tvs[r8]
                    cs_t = plsc.cumsum(m_eq.astype(jnp.int32))
                    m_acc = m_eq & ((tiec + cs_t) <= ntv)
                    m_sel = m_gt | m_acc
                    cs_s = plsc.cumsum(m_sel.astype(jnp.int32))
                    pos = selc + cs_s - 1
                    plsc.store_scatter(sbuf.at[p, r8], [pos], fidx,
                                       mask=m_sel)
                    return selc + jnp.max(cs_s), tiec + jnp.max(cs_t)

                lax.fori_loop(0, NCHUNK, chunk_body,
                              (jnp.zeros((16,), jnp.int32),
                               jnp.zeros((16,), jnp.int32)))

        for r8 in range(8):
            pltpu.async_copy(sbuf.at[p, r8, pl.ds(0, NE)],
                             out_hbm.at[pl.ds((rows0 + r8) * NE, NE)], semo)

        @pl.when(g + 2 < NG)
        def _prefetch():
            dma_group(g + 2, p, sem)

    def outer(g2, carry):
        process_group(2 * g2, 0, sem0, semo0)
        process_group(2 * g2 + 1, 1, sem1, semo1)
        return carry

    lax.fori_loop(0, NG // 2, outer, jnp.zeros((), jnp.int32))

    for p, g in ((0, NG - 2), (1, NG - 1)):
        rows0 = base + g * 8
        for r8 in range(8):
            pltpu.make_async_copy(
                sbuf.at[p, r8, pl.ds(0, NE)],
                out_hbm.at[pl.ds((rows0 + r8) * NE, NE)],
                semo0 if p == 0 else semo1).wait()


@functools.cache
def _sc_select_fn():
    return pl.kernel(
        _sc_body,
        compiler_params=pltpu.CompilerParams(needs_layout_passes=False),
        out_type=jax.ShapeDtypeStruct((SR * NE,), jnp.float32),
        mesh=plsc.VectorSubcoreMesh(core_axis_name="c", subcore_axis_name="s",
                                    num_cores=NC, num_subcores=NS),
        scratch_types=[
            pltpu.VMEM((2, 8, S), jnp.float32),
            pltpu.VMEM((RPT + 16,), jnp.float32),
            pltpu.VMEM((RPT + 16,), jnp.int32),
            pltpu.VMEM((RPT + 16,), jnp.int32),
            pltpu.VMEM((2, 8, 128), jnp.float32),
            pltpu.SemaphoreType.DMA,
            pltpu.SemaphoreType.DMA,
            pltpu.SemaphoreType.DMA,
            pltpu.SemaphoreType.DMA,
        ],
    )


def kernel(keys, query, Wk, Wq):
    sc = _sc_select_fn()
    nodes_sl, edges_sl = [], []
    for sl in range(NSLICE):
        bs, hs = divmod(sl, NSLICE // B)
        e, t, nt, cge, nodes = _tc_energy_threshold(keys, query, Wk, Wq,
                                                    bs, hs)
        edges = sc(e, t.reshape(SR), nt.reshape(SR), cge.reshape(SR))
        nodes_sl.append(nodes.reshape(SR * NE))
        edges_sl.append(edges)
    hb = NSLICE // B
    nodes_all = jnp.stack(
        [jnp.concatenate(nodes_sl[b * hb:(b + 1) * hb]) for b in range(B)])
    edges_all = jnp.stack(
        [jnp.concatenate(edges_sl[b * hb:(b + 1) * hb]) for b in range(B)])
    return jnp.stack([nodes_all, edges_all], axis=1)


# speedup vs baseline: 1.1335x; 1.1335x over previous
import functools

import jax
import jax.numpy as jnp
from jax import lax
from jax.experimental import pallas as pl
from jax.experimental.pallas import tpu as pltpu
from jax.experimental.pallas import tpu_sc as plsc

EMBED = 768
HEADS = 12
HD = EMBED // HEADS
NE = 96
S = 4096
B = 2
QBLK = 256
NBLK = S // QBLK

ROWS = B * S
NSLICE = 4
SR = ROWS // NSLICE
NBLK_SL = SR // QBLK
NC = 2
NS = 16
NTILES = NC * NS
RPT = SR // NTILES
LN = 16
NCHUNK = S // LN


def _make_tc_body(row0):
    def _tc_body(keys_ref, q_ref, wk_ref, wq_ref, e_ref, t_ref, nt_ref,
                 cge_ref, nodes_ref, kp_ref):
        j = pl.program_id(0)

        @pl.when(j == 0)
        def _project_keys():
            for h in range(HEADS):
                kh = keys_ref[0, :, h * HD:(h + 1) * HD]
                kp_ref[:, h * HD:(h + 1) * HD] = lax.dot_general(
                    kh, wk_ref[...], (((1,), (1,)), ((), ())),
                    preferred_element_type=jnp.float32)

        qblk = q_ref[0]
        qp = jnp.concatenate(
            [lax.dot_general(qblk[:, h * HD:(h + 1) * HD], wq_ref[...],
                             (((1,), (1,)), ((), ())),
                             preferred_element_type=jnp.float32)
             for h in range(HEADS)], axis=1)
        e = lax.dot_general(qp, kp_ref[...], (((1,), (1,)), ((), ())),
                            preferred_element_type=jnp.float32)
        e_ref[...] = e

        bits = lax.bitcast_convert_type(e, jnp.int32)
        skey = jnp.where(bits < 0, bits ^ jnp.int32(0x7FFFFFFF), bits)

        lo = jnp.full((QBLK, 1), jnp.iinfo(jnp.int32).min, dtype=jnp.int32)
        hi = jnp.full((QBLK, 1), jnp.iinfo(jnp.int32).max, dtype=jnp.int32)

        def bs_body(_, carry):
            lo, hi = carry
            extra = ((lo & 1) + (hi & 1) + 1) >> 1
            mid = (lo >> 1) + (hi >> 1) + extra
            cnt = jnp.sum((skey >= mid).astype(jnp.int32), axis=1,
                          keepdims=True)
            ge = cnt >= NE
            return jnp.where(ge, mid, lo), jnp.where(ge, hi, mid - 1)

        lo, hi = lax.fori_loop(0, 32, bs_body, (lo, hi))
        tbits = jnp.where(lo < 0, lo ^ jnp.int32(0x7FFFFFFF), lo)
        t = lax.bitcast_convert_type(tbits, jnp.float32)
        t_ref[...] = t

        strict = jnp.sum((e > t).astype(jnp.int32), axis=1, keepdims=True)
        nt_ref[...] = NE - strict
        cge_ref[...] = jnp.sum((e >= t).astype(jnp.int32), axis=1,
                               keepdims=True)

        nodes_ref[...] = (lax.broadcasted_iota(jnp.int32, (QBLK, NE), 0)
                          + (row0 + j * QBLK)).astype(jnp.float32)

    return _tc_body


def _tc_energy_threshold(keys, query, wk, wq, bs, hs):
    boff = hs * NBLK_SL
    return pl.pallas_call(
        _make_tc_body(hs * SR),
        grid=(NBLK_SL,),
        in_specs=[
            pl.BlockSpec((1, S, EMBED), lambda j: (bs, 0, 0)),
            pl.BlockSpec((1, QBLK, EMBED), lambda j: (bs, boff + j, 0)),
            pl.BlockSpec((HD, HD), lambda j: (0, 0)),
            pl.BlockSpec((HD, HD), lambda j: (0, 0)),
        ],
        out_specs=[
            pl.BlockSpec((QBLK, S), lambda j: (j, 0)),
            pl.BlockSpec((QBLK, 1), lambda j: (j, 0)),
            pl.BlockSpec((QBLK, 1), lambda j: (j, 0)),
            pl.BlockSpec((QBLK, 1), lambda j: (j, 0)),
            pl.BlockSpec((QBLK, NE), lambda j: (j, 0)),
        ],
        out_shape=[
            jax.ShapeDtypeStruct((SR, S), jnp.float32),
            jax.ShapeDtypeStruct((SR, 1), jnp.float32),
            jax.ShapeDtypeStruct((SR, 1), jnp.int32),
            jax.ShapeDtypeStruct((SR, 1), jnp.int32),
            jax.ShapeDtypeStruct((SR, NE), jnp.float32),
        ],
        scratch_shapes=[pltpu.VMEM((S, EMBED), jnp.float32)],
    )(keys, query, wk, wq)


NCB = S // 128
NG = RPT // 8


def _sc_body(e_hbm, t_hbm, nt_hbm, cge_hbm, out_hbm, rowbuf, tbuf, ntbuf,
             cgebuf, sbuf, sem0, sem1, semo0, semo1):
    c = lax.axis_index("c")
    s = lax.axis_index("s")
    wid = s * NC + c
    base = wid * RPT

    pltpu.sync_copy(t_hbm.at[pl.ds(base, RPT)], tbuf.at[pl.ds(0, RPT)])
    pltpu.sync_copy(nt_hbm.at[pl.ds(base, RPT)], ntbuf.at[pl.ds(0, RPT)])
    pltpu.sync_copy(cge_hbm.at[pl.ds(base, RPT)], cgebuf.at[pl.ds(0, RPT)])

    idx0f = lax.iota(jnp.int32, 16).astype(jnp.float32)

    def dma_group(g, p, sem):
        pltpu.async_copy(e_hbm.at[pl.ds(base + g * 8, 8)], rowbuf.at[p], sem)

    def wait_group(g, p, sem):
        pltpu.make_async_copy(e_hbm.at[pl.ds(base + g * 8, 8)],
                              rowbuf.at[p], sem).wait()

    dma_group(0, 0, sem0)
    dma_group(1, 1, sem1)

    def process_group(g, p, sem, semo):
        rows0 = base + g * 8
        wait_group(g, p, sem)

        @pl.when(g >= 2)
        def _drain_out():
            for r8 in range(8):
                pltpu.make_async_copy(
                    sbuf.at[p, r8, pl.ds(0, NE)],
                    out_hbm.at[pl.ds((rows0 - 16 + r8) * NE, NE)],
                    semo).wait()

        r0 = g * 8
        tvs = [jnp.full((16,), tbuf[pl.ds(r0 + r8, 16)][0],
                        dtype=jnp.float32) for r8 in range(8)]
        cges = [cgebuf[pl.ds(r0 + r8, 16)][0] for r8 in range(8)]
        gfast = cges[0] == NE
        for r8 in range(1, 8):
            gfast = gfast & (cges[r8] == NE)

        def _chunk_addr(i, r8):
            return r8, i * LN, idx0f + (i * LN).astype(jnp.float32)

        @pl.when(gfast)
        def _fast():
            def chunk_body(i, curs):
                out = []
                for r8 in range(8):
                    ri, joff, fidx = _chunk_addr(i, r8)
                    cur = curs[r8]
                    v = rowbuf[p, ri, pl.ds(joff, LN)]
                    m = v >= tvs[r8]
                    plsc.store_compressed(sbuf.at[p, r8, pl.ds(cur, LN)],
                                          fidx, mask=m)
                    out.append(cur + plsc.all_reduce_population_count(m)[0])
                return tuple(out)

            lax.fori_loop(0, NCHUNK, chunk_body,
                          tuple(jnp.zeros((), jnp.int32) for _ in range(8)))

        @pl.when(jnp.logical_not(gfast))
        def _slow():
            for r8 in range(8):
                ntv = jnp.full((16,), ntbuf[pl.ds(r0 + r8, 16)][0],
                               dtype=jnp.int32)

                def chunk_body(i, carry):
                    selc, tiec = carry
                    ri, joff, fidx = _chunk_addr(i, r8)
                    v = rowbuf[p, ri, pl.ds(joff, LN)]
                    m_gt = v > tvs[r8]
                    m_eq = v == tvs[r8]
                    cs_t = plsc.cumsum(m_eq.astype(jnp.int32))
                    m_acc = m_eq & ((tiec + cs_t) <= ntv)
                    m_sel = m_gt | m_acc
                    cs_s = plsc.cumsum(m_sel.astype(jnp.int32))
                    pos = selc + cs_s - 1
                    plsc.store_scatter(sbuf.at[p, r8], [pos], fidx,
                                       mask=m_sel)
                    return selc + jnp.max(cs_s), tiec + jnp.max(cs_t)

                lax.fori_loop(0, NCHUNK, chunk_body,
                              (jnp.zeros((16,), jnp.int32),
                               jnp.zeros((16,), jnp.int32)))

        for r8 in range(8):
            pltpu.async_copy(sbuf.at[p, r8, pl.ds(0, NE)],
                             out_hbm.at[pl.ds((rows0 + r8) * NE, NE)], semo)

        @pl.when(g + 2 < NG)
        def _prefetch():
            dma_group(g + 2, p, sem)

    def outer(g2, carry):
        process_group(2 * g2, 0, sem0, semo0)
        process_group(2 * g2 + 1, 1, sem1, semo1)
        return carry

    lax.fori_loop(0, NG // 2, outer, jnp.zeros((), jnp.int32))

    for p, g in ((0, NG - 2), (1, NG - 1)):
        rows0 = base + g * 8
        for r8 in range(8):
            pltpu.make_async_copy(
                sbuf.at[p, r8, pl.ds(0, NE)],
                out_hbm.at[pl.ds((rows0 + r8) * NE, NE)],
                semo0 if p == 0 else semo1).wait()


@functools.cache
def _sc_select_fn():
    return pl.kernel(
        _sc_body,
        compiler_params=pltpu.CompilerParams(needs_layout_passes=False),
        out_type=jax.ShapeDtypeStruct((SR * NE,), jnp.float32),
        mesh=plsc.VectorSubcoreMesh(core_axis_name="c", subcore_axis_name="s",
                                    num_cores=NC, num_subcores=NS),
        scratch_types=[
            pltpu.VMEM((2, 8, S), jnp.float32),
            pltpu.VMEM((RPT + 16,), jnp.float32),
            pltpu.VMEM((RPT + 16,), jnp.int32),
            pltpu.VMEM((RPT + 16,), jnp.int32),
            pltpu.VMEM((2, 8, 128), jnp.float32),
            pltpu.SemaphoreType.DMA,
            pltpu.SemaphoreType.DMA,
            pltpu.SemaphoreType.DMA,
            pltpu.SemaphoreType.DMA,
        ],
    )


def kernel(keys, query, Wk, Wq):
    sc = _sc_select_fn()
    nodes_sl, edges_sl = [], []
    for sl in range(NSLICE):
        bs, hs = divmod(sl, NSLICE // B)
        e, t, nt, cge, nodes = _tc_energy_threshold(keys, query, Wk, Wq,
                                                    bs, hs)
        edges = sc(e, t.reshape(SR), nt.reshape(SR), cge.reshape(SR))
        nodes_sl.append(nodes.reshape(SR * NE))
        edges_sl.append(edges)
    hb = NSLICE // B
    nodes_all = jnp.stack(
        [jnp.concatenate(nodes_sl[b * hb:(b + 1) * hb]) for b in range(B)])
    edges_all = jnp.stack(
        [jnp.concatenate(edges_sl[b * hb:(b + 1) * hb]) for b in range(B)])
    return jnp.stack([nodes_all, edges_all], axis=1)
